# baseline (speedup 1.0000x reference)
import functools

import jax
import jax.numpy as jnp
from jax import lax
from jax.experimental import pallas as pl
from jax.experimental.pallas import tpu as pltpu
from jax.experimental.pallas import tpu_sc as plsc

MAXLEN = 200
VOCAB = 100000
D = 64
BATCH = 4096

NC = 2
NS = 16
NW = NC * NS
ROWS_PER_W = BATCH // NW
NBUF = 4
R1, R2 = 2, MAXLEN // 2
GROUPS = D // 16


def _body(x_hbm, tok_hbm, pos_hbm, out_hbm,
          idx_all, posv, shpos, b0, b1, b2, b3,
          g0, g1, g2, g3, s0, s1, s2, s3):
  bufs = (b0, b1, b2, b3)
  gsems = (g0, g1, g2, g3)
  ssems = (s0, s1, s2, s3)

  wid = lax.axis_index("s") * NC + lax.axis_index("c")
  row0 = wid * ROWS_PER_W

  pltpu.sync_copy(x_hbm.at[pl.ds(row0, ROWS_PER_W)], idx_all)

  @pl.when(lax.axis_index("s") == 0)
  def _():
    pltpu.sync_copy(pos_hbm, shpos)

  plsc.subcore_barrier()

  def start_gather(c, slot):
    for r1 in range(R1):
      pltpu.async_copy(tok_hbm.at[idx_all.at[c, r1]], bufs[slot].at[r1],
                       gsems[slot], add=True)

  def wait_gather(c, slot):
    for r1 in range(R1):
      pltpu.make_async_copy(tok_hbm.at[idx_all.at[c, r1]], bufs[slot].at[r1],
                            gsems[slot]).wait()

  def start_store(c, slot):
    pltpu.async_copy(bufs[slot], out_hbm.at[row0 + c], ssems[slot])

  def wait_store(slot):
    pltpu.make_async_copy(bufs[slot], out_hbm.at[row0], ssems[slot]).wait()

  pltpu.sync_copy(pos_hbm, bufs[0])
  pltpu.sync_copy(pos_hbm, bufs[1])
  start_gather(0, 0)
  start_gather(1, 1)

  def chunk(c, slot):
    wait_gather(c, slot)
    start_store(c, slot)

    c2 = c + 2
    s2_ = (slot + 2) % NBUF

    @pl.when(c2 < ROWS_PER_W)
    def _():
      @pl.when(c >= 2)
      def _():
        wait_store(s2_)
      pltpu.sync_copy(pos_hbm, bufs[s2_])
      start_gather(c2, s2_)

  @pl.loop(0, ROWS_PER_W, step=NBUF)
  def _(k):
    for b in range(NBUF):
      chunk(k + b, b)

  for b in range(NBUF):
    wait_store(b)


@jax.jit
def kernel(x, token_table, pos_table):
  x3 = x.astype(jnp.int32).reshape(BATCH, R1, R2)
  pos3 = pos_table.reshape(R1, R2, D)
  mesh = plsc.VectorSubcoreMesh(core_axis_name="c", subcore_axis_name="s")
  fn = pl.kernel(
      _body,
      out_type=jax.ShapeDtypeStruct((BATCH, R1, R2, D), jnp.float32),
      mesh=mesh,
      compiler_params=pltpu.CompilerParams(use_tc_tiling_on_sc=False),
      scratch_types=[
          pltpu.VMEM((ROWS_PER_W, R1, R2), jnp.int32),
          pltpu.VMEM((R1, R2, D), jnp.float32),
          pltpu.VMEM_SHARED((R1, R2, D), jnp.float32),
          pltpu.VMEM((R1, R2, D), jnp.float32),
          pltpu.VMEM((R1, R2, D), jnp.float32),
          pltpu.VMEM((R1, R2, D), jnp.float32),
          pltpu.VMEM((R1, R2, D), jnp.float32),
          pltpu.SemaphoreType.DMA,
          pltpu.SemaphoreType.DMA,
          pltpu.SemaphoreType.DMA,
          pltpu.SemaphoreType.DMA,
          pltpu.SemaphoreType.DMA,
          pltpu.SemaphoreType.DMA,
          pltpu.SemaphoreType.DMA,
          pltpu.SemaphoreType.DMA,
      ],
  )
  out = fn(x3, token_table, pos3)
  return out.reshape(BATCH, MAXLEN, D)


# speedup vs baseline: 1.8533x; 1.8533x over previous
import functools

import jax
import jax.numpy as jnp
from jax import lax
from jax.experimental import pallas as pl
from jax.experimental.pallas import tpu as pltpu
from jax.experimental.pallas import tpu_sc as plsc

MAXLEN = 200
VOCAB = 100000
D = 64
BATCH = 4096

NC = 2
NS = 16
NW = NC * NS
ROWS_PER_W = BATCH // NW
HALF = MAXLEN // 2
NCHUNK = ROWS_PER_W * 2
NBUF = 8
DEPTH = 6
GROUPS = D // 16


def _body(x_hbm, tok_hbm, pos_hbm, out_hbm,
          idx_all, posv, b0, b1, b2, b3, b4, b5, b6, b7,
          g0, g1, g2, g3, g4, g5, g6, g7,
          s0, s1, s2, s3, s4, s5, s6, s7):
  bufs = (b0, b1, b2, b3, b4, b5, b6, b7)
  gsems = (g0, g1, g2, g3, g4, g5, g6, g7)
  ssems = (s0, s1, s2, s3, s4, s5, s6, s7)

  wid = lax.axis_index("s") * NC + lax.axis_index("c")
  chunk0 = wid * NCHUNK

  pltpu.sync_copy(x_hbm.at[pl.ds(chunk0, NCHUNK)], idx_all)
  pltpu.sync_copy(pos_hbm, posv)

  def start_gather(c, slot):
    pltpu.async_copy(tok_hbm.at[idx_all.at[c]], bufs[slot], gsems[slot])

  def wait_gather(c, slot):
    pltpu.make_async_copy(tok_hbm.at[idx_all.at[c]], bufs[slot],
                          gsems[slot]).wait()

  def start_store(c, slot):
    pltpu.async_copy(bufs[slot], out_hbm.at[chunk0 + c], ssems[slot])

  def wait_store(slot):
    pltpu.make_async_copy(bufs[slot], out_hbm.at[chunk0], ssems[slot]).wait()

  for c in range(DEPTH):
    start_gather(c, c)

  def chunk(c, slot):
    wait_gather(c, slot)
    buf = bufs[slot]
    h = lax.rem(c, 2)

    @plsc.parallel_loop(0, HALF, unroll=4)
    def _(r):
      for g in range(GROUPS):
        sl = pl.ds(g * 16, 16)
        buf[r, sl] = buf[r, sl] + posv[h, r, sl]

    start_store(c, slot)

    c2 = c + DEPTH
    s2_ = (slot + DEPTH) % NBUF

    @pl.when(c2 < NCHUNK)
    def _():
      @pl.when(c >= NBUF - DEPTH)
      def _():
        wait_store(s2_)
      start_gather(c2, s2_)

  @pl.loop(0, NCHUNK, step=NBUF)
  def _(k):
    for b in range(NBUF):
      chunk(k + b, b)

  for b in range(NBUF):
    wait_store(b)


@jax.jit
def kernel(x, token_table, pos_table):
  x2 = x.astype(jnp.int32).reshape(BATCH * 2, HALF)
  pos2 = pos_table.reshape(2, HALF, D)
  mesh = plsc.VectorSubcoreMesh(core_axis_name="c", subcore_axis_name="s")
  fn = pl.kernel(
      _body,
      out_type=jax.ShapeDtypeStruct((BATCH * 2, HALF, D), jnp.float32),
      mesh=mesh,
      compiler_params=pltpu.CompilerParams(use_tc_tiling_on_sc=False),
      scratch_types=(
          [pltpu.VMEM((NCHUNK, HALF), jnp.int32),
           pltpu.VMEM((2, HALF, D), jnp.float32)]
          + [pltpu.VMEM((HALF, D), jnp.float32)] * NBUF
          + [pltpu.SemaphoreType.DMA] * (2 * NBUF)
      ),
  )
  out = fn(x2, token_table, pos2)
  return out.reshape(BATCH, MAXLEN, D)
